# baseline (speedup 1.0000x reference)
import functools

import jax
import jax.numpy as jnp
from jax import lax
from jax.experimental import pallas as pl
from jax.experimental.pallas import tpu as pltpu
from jax.experimental.pallas import tpu_sc as plsc

HOURS = 24 * 7
SU, SL, TU, TL = 100.0, 0.0, 1000.0, 0.0
B, L, D = 1024, 20, 64
LL = L * L
D2 = 2 * D

NC, NS = 2, 16
NW = NC * NS
ROWS_W = (B * L) // NW
CH = 128
NCH = ROWS_W // CH
USR_W = B // NW


def _sc_gather_body(traj_hbm, tim_hbm, user_hbm, embl_hbm, embt_hbm, embu_hbm,
                    out_l, out_t, out_u,
                    idx_l, idx_t, idx_u, rows, rows_u, sem):
    wid = lax.axis_index("s") * NC + lax.axis_index("c")
    ubase = wid * USR_W

    pltpu.sync_copy(traj_hbm.at[wid], idx_l)
    pltpu.sync_copy(tim_hbm.at[wid], idx_t)
    pltpu.sync_copy(user_hbm.at[pl.ds(ubase, USR_W)], idx_u)

    for j in range(NCH):
        for k in range(CH // 16):
            sl = pl.ds(k * 16, 16)
            v = idx_l[j, sl]
            idx_l[j, sl] = v >> 1
            t = idx_t[j, sl]
            t2 = jnp.where(t == 0, HOURS, t)
            idx_t[j, sl] = t2 >> 1
    for k in range(USR_W // 16):
        sl = pl.ds(k * 16, 16)
        idx_u[sl] = idx_u[sl] >> 1

    copies = [pltpu.async_copy(
        embl_hbm.at[idx_l.at[j]], rows.at[pl.ds(j * CH, CH)], sem)
        for j in range(NCH)]
    for c in copies:
        c.wait()
    pltpu.sync_copy(rows, out_l.at[pl.ds(wid * ROWS_W, ROWS_W)])

    copies = [pltpu.async_copy(
        embt_hbm.at[idx_t.at[j]], rows.at[pl.ds(j * CH, CH)], sem)
        for j in range(NCH)]
    for c in copies:
        c.wait()
    pltpu.sync_copy(rows, out_t.at[pl.ds(wid * ROWS_W, ROWS_W)])

    pltpu.async_copy(embu_hbm.at[idx_u], rows_u, sem).wait()
    pltpu.sync_copy(rows_u, out_u.at[pl.ds(ubase, USR_W)])


@functools.cache
def _sc_gather_kernel():
    mesh = plsc.VectorSubcoreMesh(
        core_axis_name="c", subcore_axis_name="s",
        num_cores=NC, num_subcores=NS)
    return pl.kernel(
        _sc_gather_body,
        mesh=mesh,
        out_type=(
            jax.ShapeDtypeStruct((B * L, D2), jnp.float32),
            jax.ShapeDtypeStruct((B * L, D2), jnp.float32),
            jax.ShapeDtypeStruct((B, D2), jnp.float32),
        ),
        scratch_types=[
            pltpu.VMEM((NCH, CH), jnp.int32),
            pltpu.VMEM((NCH, CH), jnp.int32),
            pltpu.VMEM((USR_W,), jnp.int32),
            pltpu.VMEM((ROWS_W, D2), jnp.float32),
            pltpu.VMEM((USR_W, D2), jnp.float32),
            pltpu.SemaphoreType.DMA,
        ],
        compiler_params=pltpu.CompilerParams(use_tc_tiling_on_sc=True),
    )


BB = 8


def _half_select(pair, parity):
    lo = pair[..., :D]
    hi = pair[..., D:]
    return jnp.where(parity == 1, hi, lo)


def _tc_body(tl_ref, ds_ref, dt_ref, traj_ref, tim_ref, usr_ref,
             rl_ref, rt_ref, ru_ref,
             esl_ref, esu_ref, etl_ref, etu_ref,
             joint_ref, delta_ref):
    loc_e = _half_select(rl_ref[...], traj_ref[...] & 1)
    tim = tim_ref[...]
    tim2 = jnp.where(tim == 0, HOURS, tim)
    time_e = _half_select(rt_ref[...], tim2 & 1)
    usr_e = _half_select(ru_ref[...], usr_ref[...] & 1)
    joint_ref[...] = loc_e + time_e + usr_e[:, None, :]

    tl = tl_ref[...]
    r = lax.broadcasted_iota(jnp.int32, (BB, LL, D), 1)
    ii = r // L
    jj = r - ii * L
    m = (tl > ii) & (tl > jj)

    esl = esl_ref[...]
    esu = esu_ref[...]
    etl = etl_ref[...]
    etu = etu_ref[...]
    inv_s = 1.0 / (SU - SL)
    inv_t = 1.0 / (TU - TL)
    a = (esu - esl) * inv_s
    b = (etu - etl) * inv_t
    c = (esl * SU - esu * SL) * inv_s + (etl * TU - etu * TL) * inv_t

    wa = jnp.where(m, a[1][None, None, :], a[0][None, None, :])
    wb = jnp.where(m, b[1][None, None, :], b[0][None, None, :])
    wc = jnp.where(m, c[1][None, None, :], c[0][None, None, :])

    ds = ds_ref[...]
    dt = dt_ref[...]
    delta_ref[...] = wa * ds + wb * dt + wc


_small = pl.BlockSpec((2, D), lambda i: (0, 0))

_tc_dense = pl.pallas_call(
    _tc_body,
    grid=(B // BB,),
    in_specs=[
        pl.BlockSpec((BB, 1, 1), lambda i: (i, 0, 0)),
        pl.BlockSpec((BB, LL, 1), lambda i: (i, 0, 0)),
        pl.BlockSpec((BB, LL, 1), lambda i: (i, 0, 0)),
        pl.BlockSpec((BB, L, 1), lambda i: (i, 0, 0)),
        pl.BlockSpec((BB, L, 1), lambda i: (i, 0, 0)),
        pl.BlockSpec((BB, 1), lambda i: (i, 0)),
        pl.BlockSpec((BB, L, D2), lambda i: (i, 0, 0)),
        pl.BlockSpec((BB, L, D2), lambda i: (i, 0, 0)),
        pl.BlockSpec((BB, D2), lambda i: (i, 0)),
        _small, _small, _small, _small,
    ],
    out_specs=(
        pl.BlockSpec((BB, L, D), lambda i: (i, 0, 0)),
        pl.BlockSpec((BB, LL, D), lambda i: (i, 0, 0)),
    ),
    out_shape=(
        jax.ShapeDtypeStruct((B, L, D), jnp.float32),
        jax.ShapeDtypeStruct((B, LL, D), jnp.float32),
    ),
    compiler_params=pltpu.CompilerParams(
        dimension_semantics=("arbitrary",)),
)


def kernel(user, tim, traj, mat, traj_len, emb_t, emb_l, emb_u,
           emb_su, emb_sl, emb_tu, emb_tl):
    traj3d = traj.astype(jnp.int32).reshape(NW, NCH, CH)
    tim3d = tim.astype(jnp.int32).reshape(NW, NCH, CH)
    user_i = user.astype(jnp.int32)

    embl2 = emb_l.reshape(emb_l.shape[0] // 2, D2)
    embt2 = jnp.pad(emb_t, ((0, 1), (0, 0))).reshape((emb_t.shape[0] + 1) // 2, D2)
    embu2 = emb_u.reshape(emb_u.shape[0] // 2, D2)

    rows_l, rows_t, rows_u = _sc_gather_kernel()(
        traj3d, tim3d, user_i, embl2, embt2, embu2)

    ds2 = mat[:, :, :, 0].reshape(B, LL, 1)
    dt2 = mat[:, :, :, 1].reshape(B, LL, 1)
    tl2 = traj_len.astype(jnp.int32).reshape(B, 1, 1)

    joint, delta3 = _tc_dense(
        tl2, ds2, dt2,
        traj.astype(jnp.int32).reshape(B, L, 1),
        tim.astype(jnp.int32).reshape(B, L, 1),
        user_i.reshape(B, 1),
        rows_l.reshape(B, L, D2), rows_t.reshape(B, L, D2), rows_u,
        emb_sl, emb_su, emb_tl, emb_tu)

    return joint, delta3.reshape(B, L, L, D)


# speedup vs baseline: 1.6982x; 1.6982x over previous
import functools

import jax
import jax.numpy as jnp
from jax import lax
from jax.experimental import pallas as pl
from jax.experimental.pallas import tpu as pltpu
from jax.experimental.pallas import tpu_sc as plsc

HOURS = 24 * 7
SU, SL, TU, TL = 100.0, 0.0, 1000.0, 0.0
B, L, D = 1024, 20, 64
LL = L * L

NC, NS = 2, 16
NW = NC * NS
ROWS_W = (B * L) // NW
CH = 128
NCH = ROWS_W // CH
USR_W = B // NW


def _sc_gather_body(traj_hbm, tim_hbm, user_hbm, embl_hbm, embt_hbm, embu_hbm,
                    out_l, out_t, out_u,
                    idx_l, idx_t, idx_u, rows_l, rows_t, rows_u, sem):
    wid = lax.axis_index("s") * NC + lax.axis_index("c")
    ubase = wid * USR_W

    pltpu.sync_copy(traj_hbm.at[wid], idx_l)
    pltpu.sync_copy(tim_hbm.at[wid], idx_t)
    pltpu.sync_copy(user_hbm.at[pl.ds(ubase, USR_W)], idx_u)

    for j in range(NCH):
        for k in range(CH // 16):
            sl = pl.ds(k * 16, 16)
            v = idx_t[j, sl]
            idx_t[j, sl] = jnp.where(v == 0, HOURS, v)

    copies = []
    for j in range(NCH):
        copies.append(pltpu.async_copy(
            embl_hbm.at[idx_l.at[j]], rows_l.at[pl.ds(j * CH, CH)], sem))
    for j in range(NCH):
        copies.append(pltpu.async_copy(
            embt_hbm.at[idx_t.at[j]], rows_t.at[pl.ds(j * CH, CH)], sem))
    copies.append(pltpu.async_copy(embu_hbm.at[idx_u], rows_u, sem))
    for c in copies:
        c.wait()

    pltpu.sync_copy(rows_l, out_l.at[pl.ds(wid * ROWS_W, ROWS_W)])
    pltpu.sync_copy(rows_t, out_t.at[pl.ds(wid * ROWS_W, ROWS_W)])
    pltpu.sync_copy(rows_u, out_u.at[pl.ds(ubase, USR_W)])


@functools.cache
def _sc_gather_kernel():
    mesh = plsc.VectorSubcoreMesh(
        core_axis_name="c", subcore_axis_name="s",
        num_cores=NC, num_subcores=NS)
    return pl.kernel(
        _sc_gather_body,
        mesh=mesh,
        out_type=(
            jax.ShapeDtypeStruct((B * L, D), jnp.float32),
            jax.ShapeDtypeStruct((B * L, D), jnp.float32),
            jax.ShapeDtypeStruct((B, D), jnp.float32),
        ),
        scratch_types=[
            pltpu.VMEM((NCH, CH), jnp.int32),
            pltpu.VMEM((NCH, CH), jnp.int32),
            pltpu.VMEM((USR_W,), jnp.int32),
            pltpu.VMEM((ROWS_W, D), jnp.float32),
            pltpu.VMEM((ROWS_W, D), jnp.float32),
            pltpu.VMEM((USR_W, D), jnp.float32),
            pltpu.SemaphoreType.DMA,
        ],
        compiler_params=pltpu.CompilerParams(use_tc_tiling_on_sc=False),
    )


I_BLK = 2


def _delta_body(tl_ref, mat_ref, esl_ref, esu_ref, etl_ref, etu_ref, out_ref):
    i0 = pl.program_id(0) * I_BLK
    tl = tl_ref[...]
    ii = i0 + lax.broadcasted_iota(jnp.int32, (I_BLK, L, 1, 1), 0)
    jj = lax.broadcasted_iota(jnp.int32, (I_BLK, L, 1, 1), 1)
    m = (tl > ii) & (tl > jj)

    esl = esl_ref[...]
    esu = esu_ref[...]
    etl = etl_ref[...]
    etu = etu_ref[...]
    inv_s = 1.0 / (SU - SL)
    inv_t = 1.0 / (TU - TL)
    a = (esu - esl) * inv_s
    b = (etu - etl) * inv_t
    c = (esl * SU - esu * SL) * inv_s + (etl * TU - etu * TL) * inv_t

    wa = jnp.where(m, a[:, :, :, 1:2], a[:, :, :, 0:1])
    wb = jnp.where(m, b[:, :, :, 1:2], b[:, :, :, 0:1])
    wc = jnp.where(m, c[:, :, :, 1:2], c[:, :, :, 0:1])

    ds = mat_ref[:, :, 0:1, :]
    dt = mat_ref[:, :, 1:2, :]
    out_ref[...] = wa * ds + wb * dt + wc


_full4 = lambda shape: pl.BlockSpec(shape, lambda i: (0, 0, 0, 0))

_tc_delta = pl.pallas_call(
    _delta_body,
    grid=(L // I_BLK,),
    in_specs=[
        _full4((1, 1, 1, B)),
        pl.BlockSpec((I_BLK, L, 2, B), lambda i: (i, 0, 0, 0)),
        _full4((1, 1, D, 2)), _full4((1, 1, D, 2)),
        _full4((1, 1, D, 2)), _full4((1, 1, D, 2)),
    ],
    out_specs=pl.BlockSpec((I_BLK, L, D, B), lambda i: (i, 0, 0, 0)),
    out_shape=jax.ShapeDtypeStruct((L, L, D, B), jnp.float32),
    compiler_params=pltpu.CompilerParams(
        dimension_semantics=("arbitrary",)),
)


BBJ = 32


def _joint_body(rl_ref, rt_ref, ru_ref, joint_ref):
    joint_ref[...] = rl_ref[...] + rt_ref[...] + ru_ref[...][:, None, :]


_tc_joint = pl.pallas_call(
    _joint_body,
    grid=(B // BBJ,),
    in_specs=[
        pl.BlockSpec((BBJ, L, D), lambda i: (i, 0, 0)),
        pl.BlockSpec((BBJ, L, D), lambda i: (i, 0, 0)),
        pl.BlockSpec((BBJ, D), lambda i: (i, 0)),
    ],
    out_specs=pl.BlockSpec((BBJ, L, D), lambda i: (i, 0, 0)),
    out_shape=jax.ShapeDtypeStruct((B, L, D), jnp.float32),
    compiler_params=pltpu.CompilerParams(
        dimension_semantics=("arbitrary",)),
)


def kernel(user, tim, traj, mat, traj_len, emb_t, emb_l, emb_u,
           emb_su, emb_sl, emb_tu, emb_tl):
    traj3d = traj.astype(jnp.int32).reshape(NW, NCH, CH)
    tim3d = tim.astype(jnp.int32).reshape(NW, NCH, CH)
    user_i = user.astype(jnp.int32)

    mat_p = jnp.transpose(mat, (1, 2, 3, 0))
    tl4 = traj_len.astype(jnp.int32).reshape(1, 1, 1, B)
    esl_p = emb_sl.T.reshape(1, 1, D, 2)
    esu_p = emb_su.T.reshape(1, 1, D, 2)
    etl_p = emb_tl.T.reshape(1, 1, D, 2)
    etu_p = emb_tu.T.reshape(1, 1, D, 2)
    delta_p = _tc_delta(tl4, mat_p, esl_p, esu_p, etl_p, etu_p)
    delta = jnp.transpose(delta_p, (3, 0, 1, 2))

    rows_l, rows_t, rows_u = _sc_gather_kernel()(
        traj3d, tim3d, user_i, emb_l, emb_t, emb_u)

    joint = _tc_joint(
        rows_l.reshape(B, L, D), rows_t.reshape(B, L, D), rows_u)

    return joint, delta
